# Initial kernel scaffold; baseline (speedup 1.0000x reference)
import jax
import jax.numpy as jnp
from jax.experimental import pallas as pl


def kernel(x, rehearsal, attn_W1, attn_b1, attn_W2, attn_b2, bag_W, bag_b, dsl_W1, dsl_b1, dsl_W2, dsl_b2, hgc1_W, hgc1_att, hgc1_b, hgc2_W, hgc2_att, hgc2_b, gn1_w, gn1_b, gn1_ms, gn2_w, gn2_b, gn2_ms, fc1_W, fc1_b, fc2_W, fc2_b, gat_W1, gat_b1, gat_W2, gat_b2, cls_W, cls_b):
    raise NotImplementedError("write your pallas kernel here")


# speedup vs baseline: 14.6755x; 14.6755x over previous
import jax
import jax.numpy as jnp
from jax import lax
from jax.experimental import pallas as pl

_FEAT = 512
_HID = 256
_NC = 2
_K = 8
_BUF = 256
_BATCH = 4
_NINST = 4096


def _dot(a, b):
    return lax.dot_general(a, b, (((1,), (0,)), ((), ())),
                           preferred_element_type=jnp.float32)


def _dot_nt(a, b):
    return lax.dot_general(a, b, (((1,), (1,)), ((), ())),
                           preferred_element_type=jnp.float32)


def _dot_tn(a, b):
    return lax.dot_general(a, b, (((0,), (0,)), ((), ())),
                           preferred_element_type=jnp.float32)


def _leaky(x, ns):
    return jnp.where(x >= 0, x, ns * x)


def _attn_kernel(x_ref, w1_ref, b1_ref, w2_ref, b2_ref, m_ref):
    xb = x_ref[0]
    h = jnp.maximum(_dot_nt(xb, w1_ref[...]) + b1_ref[...], 0.0)
    a = _dot_nt(h, w2_ref[...])[:, 0:1] + b2_ref[0, 0]
    amax = jnp.max(a)
    e = jnp.exp(a - amax)
    p = e / jnp.sum(e)
    m_ref[0] = _dot_tn(p, xb)


def _hgconv_dense(xin, edge_attr, E, dinv, W, att1, att2, bias):
    xl = _dot_nt(xin, W)
    ha = _dot_nt(edge_attr, W)
    u = _dot_nt(att1, xl)
    v = _dot_nt(ha, att2)[:, 0:1]
    L = _leaky(v + u, 0.2)
    on = E > 0.0
    Lm = jnp.where(on, L, -jnp.inf)
    amax = jnp.max(Lm, axis=0, keepdims=True)
    amax = jnp.where(amax == -jnp.inf, 0.0, amax)
    A = jnp.where(on, jnp.exp(L - amax), 0.0)
    asum = jnp.sum(A, axis=0, keepdims=True)
    alpha = A / (asum + 1e-16)
    eout = _dot(alpha, xl) * 0.125
    outn = dinv * _dot_tn(alpha, eout)
    return outn + bias


def _graphnorm(x, w, b, ms, eps=1e-5):
    mean = jnp.mean(x, axis=0, keepdims=True)
    out = x - mean * ms
    var = jnp.mean(out * out, axis=0, keepdims=True)
    return w * out / jnp.sqrt(var + eps) + b


def _graph_kernel(xc_ref,
                  dw1_ref, db1_ref, dw2_ref, db2_ref,
                  h1w_ref, h1a1_ref, h1a2_ref, h1b_ref,
                  h2w_ref, h2a1_ref, h2a2_ref, h2b_ref,
                  g1w_ref, g1b_ref, g1m_ref,
                  g2w_ref, g2b_ref, g2m_ref,
                  f1w_ref, f1b_ref, f2w_ref, f2b_ref,
                  gw1_ref, gb1_ref, gw2_ref, gb2_ref,
                  clsw_ref, clsb_ref, bagw_ref, bagb_ref,
                  mlp_ref, graph_ref):
    xc = xc_ref[...]

    mlp_ref[...] = _dot_nt(xc[0:8], bagw_ref[...]) + bagb_ref[...]

    xd = _leaky(_dot_nt(xc, dw1_ref[...]) + db1_ref[...], 0.01)
    xd = _leaky(_dot_nt(xd, dw2_ref[...]) + db2_ref[...], 0.01)

    nrm = jnp.sqrt(jnp.sum(xd * xd, axis=1, keepdims=True))
    xn = xd / (nrm + 1e-12)
    sim = _dot_nt(xn, xn)
    col = lax.broadcasted_iota(jnp.int32, (_BUF, _BUF), 1)
    E = jnp.zeros((_BUF, _BUF), jnp.float32)
    work = sim
    for _ in range(_K):
        rmax = jnp.max(work, axis=1, keepdims=True)
        cand = jnp.where(work == rmax, col, _BUF)
        idx = jnp.min(cand, axis=1, keepdims=True)
        sel = col == idx
        E = E + jnp.where(sel, 1.0, 0.0)
        work = jnp.where(sel, -jnp.inf, work)

    edge_attr = _dot(E, xd) * 0.125

    ones_bn = jnp.ones((_BUF, 128), jnp.float32)
    dinv = _dot_tn(E, ones_bn)[:, 0:1]
    dinv = jnp.where(dinv > 0, 1.0 / dinv, 0.0)

    h = _hgconv_dense(xd, edge_attr, E, dinv, h1w_ref[...],
                      h1a1_ref[...], h1a2_ref[...], h1b_ref[...])
    h = _leaky(_graphnorm(h, g1w_ref[...], g1b_ref[...], g1m_ref[...]), 0.01)
    out1 = _leaky(_dot_nt(h, f1w_ref[...]) + f1b_ref[...], 0.01)

    h = _hgconv_dense(h, edge_attr, E, dinv, h2w_ref[...],
                      h2a1_ref[...], h2a2_ref[...], h2b_ref[...])
    h = _leaky(_graphnorm(h, g2w_ref[...], g2b_ref[...], g2m_ref[...]), 0.01)
    out2 = _leaky(_dot_nt(h, f2w_ref[...]) + f2b_ref[...], 0.01)

    out = jnp.concatenate([xd, out1, out2], axis=1)

    t = jnp.maximum(_dot(gw1_ref[...], out) + gb1_ref[...], 0.0)
    sv = _dot(gw2_ref[...], t) + gb2_ref[0, 0]
    sv = jax.nn.sigmoid(sv)
    s = sv - jnp.mean(sv)

    graph_ref[...] = _dot_nt(out[0:8] * s, clsw_ref[...]) + clsb_ref[...]


def kernel(x, rehearsal, attn_W1, attn_b1, attn_W2, attn_b2, bag_W, bag_b,
           dsl_W1, dsl_b1, dsl_W2, dsl_b2,
           hgc1_W, hgc1_att, hgc1_b, hgc2_W, hgc2_att, hgc2_b,
           gn1_w, gn1_b, gn1_ms, gn2_w, gn2_b, gn2_ms,
           fc1_W, fc1_b, fc2_W, fc2_b,
           gat_W1, gat_b1, gat_W2, gat_b2, cls_W, cls_b):
    f32 = jnp.float32

    attn_W2p = jnp.zeros((128, _FEAT), f32).at[0].set(attn_W2[0])
    M = pl.pallas_call(
        _attn_kernel,
        grid=(_BATCH,),
        in_specs=[
            pl.BlockSpec((1, _NINST, _FEAT), lambda b: (b, 0, 0)),
            pl.BlockSpec((_FEAT, _FEAT), lambda b: (0, 0)),
            pl.BlockSpec((1, _FEAT), lambda b: (0, 0)),
            pl.BlockSpec((128, _FEAT), lambda b: (0, 0)),
            pl.BlockSpec((1, 1), lambda b: (0, 0)),
        ],
        out_specs=pl.BlockSpec((1, 1, _FEAT), lambda b: (b, 0, 0)),
        out_shape=jax.ShapeDtypeStruct((_BATCH, 1, _FEAT), f32),
    )(x, attn_W1, attn_b1.reshape(1, _FEAT), attn_W2p, attn_b2.reshape(1, 1))
    M = M.reshape(_BATCH, _FEAT)

    xc = jnp.concatenate([M, rehearsal], axis=0)[:_BUF]

    OUTD = _FEAT + 2 * _HID
    h1a2p = jnp.zeros((128, _FEAT), f32).at[0].set(hgc1_att[_FEAT:])
    h2a2p = jnp.zeros((128, _FEAT), f32).at[0].set(hgc2_att[_FEAT:])
    bag_Wp = jnp.zeros((128, _FEAT), f32).at[:_NC].set(bag_W)
    bag_bp = jnp.zeros((1, 128), f32).at[0, :_NC].set(bag_b)
    cls_Wp = jnp.zeros((128, OUTD), f32).at[:_NC].set(cls_W)
    cls_bp = jnp.zeros((1, 128), f32).at[0, :_NC].set(cls_b)

    out_shapes = (
        jax.ShapeDtypeStruct((8, 128), f32),
        jax.ShapeDtypeStruct((8, 128), f32),
    )
    logits_mlp_p, logits_graph_p = pl.pallas_call(
        _graph_kernel,
        out_shape=out_shapes,
    )(xc,
      dsl_W1, dsl_b1.reshape(1, _HID), dsl_W2, dsl_b2.reshape(1, _FEAT),
      hgc1_W, hgc1_att[: _FEAT].reshape(1, _FEAT), h1a2p, hgc1_b.reshape(1, _FEAT),
      hgc2_W, hgc2_att[: _FEAT].reshape(1, _FEAT), h2a2p, hgc2_b.reshape(1, _FEAT),
      gn1_w.reshape(1, _FEAT), gn1_b.reshape(1, _FEAT), gn1_ms.reshape(1, _FEAT),
      gn2_w.reshape(1, _FEAT), gn2_b.reshape(1, _FEAT), gn2_ms.reshape(1, _FEAT),
      fc1_W, fc1_b.reshape(1, _HID), fc2_W, fc2_b.reshape(1, _HID),
      gat_W1, gat_b1.reshape(_BUF, 1), gat_W2, gat_b2.reshape(1, 1),
      cls_Wp, cls_bp, bag_Wp, bag_bp)

    logits_mlp = logits_mlp_p[:_BATCH, :_NC]
    logits_graph = logits_graph_p[:_BATCH, :_NC]
    return logits_mlp, logits_graph


# speedup vs baseline: 17.2466x; 1.1752x over previous
import jax
import jax.numpy as jnp
from jax import lax
from jax.experimental import pallas as pl
from jax.experimental.pallas import tpu as pltpu

_FEAT = 512
_HID = 256
_NC = 2
_K = 8
_BUF = 256
_BATCH = 4
_NINST = 4096


def _dot(a, b):
    return lax.dot_general(a, b, (((1,), (0,)), ((), ())),
                           preferred_element_type=jnp.float32)


def _dot_nt(a, b):
    return lax.dot_general(a, b, (((1,), (1,)), ((), ())),
                           preferred_element_type=jnp.float32)


def _dot_tn(a, b):
    return lax.dot_general(a, b, (((0,), (0,)), ((), ())),
                           preferred_element_type=jnp.float32)


def _leaky(x, ns):
    return jnp.where(x >= 0, x, ns * x)


def _hgconv_dense(xin, edge_attr, E, dinv, W, att1, att2, bias):
    xl = _dot_nt(xin, W)
    ha = _dot_nt(edge_attr, W)
    u = _dot_nt(att1, xl)
    v = jnp.sum(ha * att2, axis=1, keepdims=True)
    L = _leaky(v + u, 0.2)
    on = E > 0.0
    Lm = jnp.where(on, L, -jnp.inf)
    amax = jnp.max(Lm, axis=0, keepdims=True)
    amax = jnp.where(amax == -jnp.inf, 0.0, amax)
    A = jnp.where(on, jnp.exp(L - amax), 0.0)
    asum = jnp.sum(A, axis=0, keepdims=True)
    alpha = A / (asum + 1e-16)
    eout = _dot(alpha, xl) * 0.125
    outn = dinv * _dot_tn(alpha, eout)
    return outn + bias


def _graphnorm(x, w, b, ms, eps=1e-5):
    mean = jnp.mean(x, axis=0, keepdims=True)
    out = x - mean * ms
    var = jnp.mean(out * out, axis=0, keepdims=True)
    return w * out / jnp.sqrt(var + eps) + b


def _fused_kernel(x_ref, reh_ref,
                  w1_ref, b1_ref, w2_ref, b2_ref,
                  dw1_ref, db1_ref, dw2_ref, db2_ref,
                  h1w_ref, h1a1_ref, h1a2_ref, h1b_ref,
                  h2w_ref, h2a1_ref, h2a2_ref, h2b_ref,
                  g1w_ref, g1b_ref, g1m_ref,
                  g2w_ref, g2b_ref, g2m_ref,
                  f1w_ref, f1b_ref, f2w_ref, f2b_ref,
                  gw1_ref, gb1_ref, gw2_ref, gb2_ref,
                  clsw_ref, clsb_ref, bagw_ref, bagb_ref,
                  mlp_ref, graph_ref, xc_scr):
    b = pl.program_id(0)

    xb = x_ref[0]
    h = jnp.maximum(_dot_nt(xb, w1_ref[...]) + b1_ref[...], 0.0)
    a = jnp.sum(h * w2_ref[...], axis=1, keepdims=True) + b2_ref[0, 0]
    amax = jnp.max(a)
    e = jnp.exp(a - amax)
    p = e / jnp.sum(e)
    xc_scr[pl.ds(b, 1), :] = _dot_tn(p, xb)

    @pl.when(b == _BATCH - 1)
    def _():
        xc_scr[_BATCH:, :] = reh_ref[0:_BUF - _BATCH, :]
        xc = xc_scr[...]

        mlp_ref[...] = _dot_nt(xc[0:8], bagw_ref[...]) + bagb_ref[...]

        xd = _leaky(_dot_nt(xc, dw1_ref[...]) + db1_ref[...], 0.01)
        xd = _leaky(_dot_nt(xd, dw2_ref[...]) + db2_ref[...], 0.01)

        nrm = jnp.sqrt(jnp.sum(xd * xd, axis=1, keepdims=True))
        xn = xd / (nrm + 1e-12)
        sim = _dot_nt(xn, xn)
        col = lax.broadcasted_iota(jnp.int32, (_BUF, _BUF), 1)
        E = jnp.zeros((_BUF, _BUF), jnp.float32)
        work = sim
        for _ in range(_K):
            rmax = jnp.max(work, axis=1, keepdims=True)
            cand = jnp.where(work == rmax, col, _BUF)
            idx = jnp.min(cand, axis=1, keepdims=True)
            sel = col == idx
            E = E + jnp.where(sel, 1.0, 0.0)
            work = jnp.where(sel, -jnp.inf, work)

        edge_attr = _dot(E, xd) * 0.125

        ones_bn = jnp.ones((_BUF, 128), jnp.float32)
        dinv = _dot_tn(E, ones_bn)[:, 0:1]
        dinv = jnp.where(dinv > 0, 1.0 / dinv, 0.0)

        hh = _hgconv_dense(xd, edge_attr, E, dinv, h1w_ref[...],
                           h1a1_ref[...], h1a2_ref[...], h1b_ref[...])
        hh = _leaky(_graphnorm(hh, g1w_ref[...], g1b_ref[...], g1m_ref[...]),
                    0.01)
        out1 = _leaky(_dot_nt(hh, f1w_ref[...]) + f1b_ref[...], 0.01)

        hh = _hgconv_dense(hh, edge_attr, E, dinv, h2w_ref[...],
                           h2a1_ref[...], h2a2_ref[...], h2b_ref[...])
        hh = _leaky(_graphnorm(hh, g2w_ref[...], g2b_ref[...], g2m_ref[...]),
                    0.01)
        out2 = _leaky(_dot_nt(hh, f2w_ref[...]) + f2b_ref[...], 0.01)

        out = jnp.concatenate([xd, out1, out2], axis=1)

        t = jnp.maximum(_dot(gw1_ref[...], out) + gb1_ref[...], 0.0)
        sv = _dot(gw2_ref[...], t) + gb2_ref[0, 0]
        sv = jax.nn.sigmoid(sv)
        s = sv - jnp.mean(sv)

        graph_ref[...] = _dot_nt(out[0:8] * s, clsw_ref[...]) + clsb_ref[...]


def kernel(x, rehearsal, attn_W1, attn_b1, attn_W2, attn_b2, bag_W, bag_b,
           dsl_W1, dsl_b1, dsl_W2, dsl_b2,
           hgc1_W, hgc1_att, hgc1_b, hgc2_W, hgc2_att, hgc2_b,
           gn1_w, gn1_b, gn1_ms, gn2_w, gn2_b, gn2_ms,
           fc1_W, fc1_b, fc2_W, fc2_b,
           gat_W1, gat_b1, gat_W2, gat_b2, cls_W, cls_b):
    f32 = jnp.float32
    OUTD = _FEAT + 2 * _HID

    bag_Wp = jnp.zeros((128, _FEAT), f32).at[:_NC].set(bag_W)
    bag_bp = jnp.zeros((1, 128), f32).at[0, :_NC].set(bag_b)
    cls_Wp = jnp.zeros((128, OUTD), f32).at[:_NC].set(cls_W)
    cls_bp = jnp.zeros((1, 128), f32).at[0, :_NC].set(cls_b)

    full = lambda b: tuple([0] * len(b))
    def spec(shape):
        return pl.BlockSpec(shape, lambda b, _s=shape: (0,) * len(_s))

    in_specs = [
        pl.BlockSpec((1, _NINST, _FEAT), lambda b: (b, 0, 0)),
        spec((_BUF, _FEAT)),
        spec((_FEAT, _FEAT)), spec((1, _FEAT)),
        spec((1, _FEAT)), spec((1, 1)),
        spec((_HID, _FEAT)), spec((1, _HID)),
        spec((_FEAT, _HID)), spec((1, _FEAT)),
        spec((_FEAT, _FEAT)), spec((1, _FEAT)), spec((1, _FEAT)),
        spec((1, _FEAT)),
        spec((_FEAT, _FEAT)), spec((1, _FEAT)), spec((1, _FEAT)),
        spec((1, _FEAT)),
        spec((1, _FEAT)), spec((1, _FEAT)), spec((1, _FEAT)),
        spec((1, _FEAT)), spec((1, _FEAT)), spec((1, _FEAT)),
        spec((_HID, _FEAT)), spec((1, _HID)),
        spec((_HID, _FEAT)), spec((1, _HID)),
        spec((_BUF, _BUF)), spec((_BUF, 1)),
        spec((1, _BUF)), spec((1, 1)),
        spec((128, OUTD)), spec((1, 128)),
        spec((128, _FEAT)), spec((1, 128)),
    ]
    out_specs = (spec((8, 128)), spec((8, 128)))
    out_shapes = (jax.ShapeDtypeStruct((8, 128), f32),
                  jax.ShapeDtypeStruct((8, 128), f32))

    logits_mlp_p, logits_graph_p = pl.pallas_call(
        _fused_kernel,
        grid=(_BATCH,),
        in_specs=in_specs,
        out_specs=out_specs,
        out_shape=out_shapes,
        scratch_shapes=[pltpu.VMEM((_BUF, _FEAT), f32)],
    )(x, rehearsal,
      attn_W1, attn_b1.reshape(1, _FEAT), attn_W2, attn_b2.reshape(1, 1),
      dsl_W1, dsl_b1.reshape(1, _HID), dsl_W2, dsl_b2.reshape(1, _FEAT),
      hgc1_W, hgc1_att[: _FEAT].reshape(1, _FEAT),
      hgc1_att[_FEAT:].reshape(1, _FEAT), hgc1_b.reshape(1, _FEAT),
      hgc2_W, hgc2_att[: _FEAT].reshape(1, _FEAT),
      hgc2_att[_FEAT:].reshape(1, _FEAT), hgc2_b.reshape(1, _FEAT),
      gn1_w.reshape(1, _FEAT), gn1_b.reshape(1, _FEAT), gn1_ms.reshape(1, _FEAT),
      gn2_w.reshape(1, _FEAT), gn2_b.reshape(1, _FEAT), gn2_ms.reshape(1, _FEAT),
      fc1_W, fc1_b.reshape(1, _HID), fc2_W, fc2_b.reshape(1, _HID),
      gat_W1, gat_b1.reshape(_BUF, 1), gat_W2, gat_b2.reshape(1, 1),
      cls_Wp, cls_bp, bag_Wp, bag_bp)

    logits_mlp = logits_mlp_p[:_BATCH, :_NC]
    logits_graph = logits_graph_p[:_BATCH, :_NC]
    return logits_mlp, logits_graph
